# baseline (speedup 1.0000x reference)
import functools

import jax
import jax.numpy as jnp
from jax import lax
from jax.experimental import pallas as pl
from jax.experimental.pallas import tpu as pltpu
from jax.experimental.pallas import tpu_sc as plsc

DIM = 64
EPS = 1e-05
L16 = 16
CHUNK = 512
GSLICE = 128


def _rsqrt(x):
    i = plsc.bitcast(x, jnp.int32)
    i = jnp.int32(0x5F3759DF) - (i >> 1)
    y = plsc.bitcast(i, jnp.float32)
    for _ in range(4):
        y = y * (1.5 - 0.5 * x * y * y)
    return y


@functools.lru_cache(maxsize=None)
def _build(total):
    info = plsc.get_sparse_core_info()
    nc, ns = info.num_cores, info.num_subcores
    nw = nc * ns
    per_w = total // nw
    assert total % (nw * CHUNK) == 0
    nchunk = per_w // CHUNK

    @functools.partial(
        pl.kernel,
        out_type=jax.ShapeDtypeStruct((total, DIM), jnp.float32),
        mesh=plsc.VectorSubcoreMesh(core_axis_name="c", subcore_axis_name="s"),
        scratch_types=[
            pltpu.VMEM((per_w,), jnp.int32),
            pltpu.VMEM((CHUNK, DIM), jnp.float32),
            pltpu.VMEM((DIM,), jnp.float32),
            pltpu.VMEM((DIM,), jnp.float32),
            pltpu.SemaphoreType.DMA,
        ],
        compiler_params=pltpu.CompilerParams(
            needs_layout_passes=False, use_tc_tiling_on_sc=False),
    )
    def sc_kernel(ids_hbm, table_hbm, w_hbm, b_hbm, out_hbm,
                  idx_v, rows_v, w_v, b_v, sem):
        wid = lax.axis_index("s") * nc + lax.axis_index("c")
        base = wid * per_w
        pltpu.sync_copy(ids_hbm.at[pl.ds(base, per_w)], idx_v)
        pltpu.sync_copy(w_hbm, w_v)
        pltpu.sync_copy(b_hbm, b_v)
        wregs = [w_v[pl.ds(k * L16, L16)] for k in range(DIM // L16)]
        bregs = [b_v[pl.ds(k * L16, L16)] for k in range(DIM // L16)]
        ri = lax.iota(jnp.int32, 16)

        def chunk_body(t, carry):
            copies = [
                pltpu.async_copy(
                    table_hbm.at[idx_v.at[pl.ds(t * CHUNK + j * GSLICE, GSLICE)]],
                    rows_v.at[pl.ds(j * GSLICE, GSLICE)],
                    sem,
                )
                for j in range(CHUNK // GSLICE)
            ]
            for cp in copies:
                cp.wait()

            def block_body(blk, c2):
                rows_idx = blk * L16 + ri
                s = jnp.zeros((L16,), jnp.float32)
                q = jnp.zeros((L16,), jnp.float32)
                for c in range(DIM):
                    cc = jnp.full((L16,), c, jnp.int32)
                    v = plsc.load_gather(rows_v, [rows_idx, cc])
                    s = s + v
                    q = q + v * v
                mean = s * (1.0 / DIM)
                var = q * (1.0 / DIM) - mean * mean
                rstd = _rsqrt(jnp.maximum(var, 0.0) + EPS)
                for c in range(DIM):
                    cc = jnp.full((L16,), c, jnp.int32)
                    v = plsc.load_gather(rows_v, [rows_idx, cc])
                    o = (v - mean) * rstd
                    plsc.store_scatter(rows_v, [rows_idx, cc], o)
                for r in range(L16):
                    rr = jnp.full((L16,), 1, jnp.int32) * (blk * L16 + r)
                    for k in range(DIM // L16):
                        cidx = ri + k * L16
                        v = plsc.load_gather(rows_v, [rr, cidx])
                        o = v * wregs[k] + bregs[k]
                        plsc.store_scatter(rows_v, [rr, cidx], o)
                return c2

            lax.fori_loop(0, CHUNK // L16, block_body, 0)
            pltpu.sync_copy(rows_v, out_hbm.at[pl.ds(base + t * CHUNK, CHUNK)])
            return carry

        lax.fori_loop(0, nchunk, chunk_body, 0)

    return sc_kernel


def kernel(input_ids, value_table, ln_weight, ln_bias):
    b, l = input_ids.shape
    ids_flat = input_ids.reshape(-1).astype(jnp.int32)
    out = _build(b * l)(ids_flat, value_table,
                        ln_weight.astype(jnp.float32),
                        ln_bias.astype(jnp.float32))
    return out.reshape(b, l, DIM)


# speedup vs baseline: 1.7417x; 1.7417x over previous
import functools

import jax
import jax.numpy as jnp
from jax import lax
from jax.experimental import pallas as pl
from jax.experimental.pallas import tpu as pltpu
from jax.experimental.pallas import tpu_sc as plsc

DIM = 64
EPS = 1e-05
L16 = 16
CHUNK = 512
GSLICE = 128


def _rsqrt(x):
    i = plsc.bitcast(x, jnp.int32)
    i = jnp.int32(0x5F3759DF) - (i >> 1)
    y = plsc.bitcast(i, jnp.float32)
    for _ in range(4):
        y = y * (1.5 - 0.5 * x * y * y)
    return y


@functools.lru_cache(maxsize=None)
def _build(total):
    info = plsc.get_sparse_core_info()
    nc, ns = info.num_cores, info.num_subcores
    nw = nc * ns
    per_w = total // nw
    assert total % (nw * CHUNK) == 0
    nchunk = per_w // CHUNK

    @functools.partial(
        pl.kernel,
        out_type=jax.ShapeDtypeStruct((total, DIM), jnp.float32),
        mesh=plsc.VectorSubcoreMesh(core_axis_name="c", subcore_axis_name="s"),
        scratch_types=[
            pltpu.VMEM((per_w,), jnp.int32),
            pltpu.VMEM((CHUNK, DIM), jnp.float32),
            pltpu.VMEM((DIM,), jnp.float32),
            pltpu.VMEM((DIM,), jnp.float32),
            pltpu.SemaphoreType.DMA,
        ],
        compiler_params=pltpu.CompilerParams(
            needs_layout_passes=False, use_tc_tiling_on_sc=False),
    )
    def sc_kernel(ids_hbm, table_hbm, w_hbm, b_hbm, out_hbm,
                  idx_v, rows_v, w_v, b_v, sem):
        wid = lax.axis_index("s") * nc + lax.axis_index("c")
        base = wid * per_w
        pltpu.sync_copy(ids_hbm.at[pl.ds(base, per_w)], idx_v)
        pltpu.sync_copy(w_hbm, w_v)
        pltpu.sync_copy(b_hbm, b_v)
        wregs = [w_v[pl.ds(k * L16, L16)] for k in range(DIM // L16)]
        bregs = [b_v[pl.ds(k * L16, L16)] for k in range(DIM // L16)]
        ri = lax.iota(jnp.int32, 16)

        def chunk_body(t, carry):
            copies = [
                pltpu.async_copy(
                    table_hbm.at[idx_v.at[pl.ds(t * CHUNK + j * GSLICE, GSLICE)]],
                    rows_v.at[pl.ds(j * GSLICE, GSLICE)],
                    sem,
                )
                for j in range(CHUNK // GSLICE)
            ]
            for cp in copies:
                cp.wait()

            def block_body(blk, c2):
                rows_idx = blk * L16 + ri
                s = jnp.zeros((L16,), jnp.float32)
                q = jnp.zeros((L16,), jnp.float32)
                for d in range(DIM):
                    cc = (ri + d) & (DIM - 1)
                    v = plsc.load_gather(rows_v, [rows_idx, cc])
                    s = s + v
                    q = q + v * v
                mean = s * (1.0 / DIM)
                var = q * (1.0 / DIM) - mean * mean
                rstd = _rsqrt(jnp.maximum(var, 0.0) + EPS)
                for d in range(DIM):
                    cc = (ri + d) & (DIM - 1)
                    v = plsc.load_gather(rows_v, [rows_idx, cc])
                    o = (v - mean) * rstd
                    plsc.store_scatter(rows_v, [rows_idx, cc], o)
                for r in range(L16):
                    rr = jnp.full((L16,), 1, jnp.int32) * (blk * L16 + r)
                    for k in range(DIM // L16):
                        cidx = ri + k * L16
                        v = plsc.load_gather(rows_v, [rr, cidx])
                        o = v * wregs[k] + bregs[k]
                        plsc.store_scatter(rows_v, [rr, cidx], o)
                return c2

            lax.fori_loop(0, CHUNK // L16, block_body, 0)
            pltpu.sync_copy(rows_v, out_hbm.at[pl.ds(base + t * CHUNK, CHUNK)])
            return carry

        lax.fori_loop(0, nchunk, chunk_body, 0)

    return sc_kernel


def kernel(input_ids, value_table, ln_weight, ln_bias):
    b, l = input_ids.shape
    ids_flat = input_ids.reshape(-1).astype(jnp.int32)
    out = _build(b * l)(ids_flat, value_table,
                        ln_weight.astype(jnp.float32),
                        ln_bias.astype(jnp.float32))
    return out.reshape(b, l, DIM)


# speedup vs baseline: 1.7847x; 1.0247x over previous
import functools

import jax
import jax.numpy as jnp
from jax import lax
from jax.experimental import pallas as pl
from jax.experimental.pallas import tpu as pltpu
from jax.experimental.pallas import tpu_sc as plsc

DIM = 64
EPS = 1e-05
L16 = 16
CHUNK = 512
GSLICE = 128


def _rsqrt(x):
    i = plsc.bitcast(x, jnp.int32)
    i = jnp.int32(0x5F3759DF) - (i >> 1)
    y = plsc.bitcast(i, jnp.float32)
    for _ in range(4):
        y = y * (1.5 - 0.5 * x * y * y)
    return y


@functools.lru_cache(maxsize=None)
def _build(total):
    info = plsc.get_sparse_core_info()
    nc, ns = info.num_cores, info.num_subcores
    nw = nc * ns
    per_w = total // nw
    assert total % (nw * CHUNK) == 0
    nchunk = per_w // CHUNK

    @functools.partial(
        pl.kernel,
        out_type=jax.ShapeDtypeStruct((total, DIM), jnp.float32),
        mesh=plsc.VectorSubcoreMesh(core_axis_name="c", subcore_axis_name="s"),
        scratch_types=[
            pltpu.VMEM((per_w,), jnp.int32),
            pltpu.VMEM((CHUNK, DIM), jnp.float32),
            pltpu.VMEM((DIM,), jnp.float32),
            pltpu.VMEM((DIM,), jnp.float32),
            pltpu.SemaphoreType.DMA,
        ],
        compiler_params=pltpu.CompilerParams(
            needs_layout_passes=False, use_tc_tiling_on_sc=False,
            disable_bounds_checks=True),
    )
    def sc_kernel(ids_hbm, table_hbm, w_hbm, b_hbm, out_hbm,
                  idx_v, rows_v, w_v, b_v, sem):
        wid = lax.axis_index("s") * nc + lax.axis_index("c")
        base = wid * per_w
        pltpu.sync_copy(ids_hbm.at[pl.ds(base, per_w)], idx_v)
        pltpu.sync_copy(w_hbm, w_v)
        pltpu.sync_copy(b_hbm, b_v)
        wregs = [w_v[pl.ds(k * L16, L16)] for k in range(DIM // L16)]
        bregs = [b_v[pl.ds(k * L16, L16)] for k in range(DIM // L16)]
        ri = lax.iota(jnp.int32, 16)

        def chunk_body(t, carry):
            copies = [
                pltpu.async_copy(
                    table_hbm.at[idx_v.at[pl.ds(t * CHUNK + j * GSLICE, GSLICE)]],
                    rows_v.at[pl.ds(j * GSLICE, GSLICE)],
                    sem,
                )
                for j in range(CHUNK // GSLICE)
            ]
            for cp in copies:
                cp.wait()

            def block_body(blk, c2):
                rows_idx = blk * L16 + ri
                ss = [jnp.zeros((L16,), jnp.float32) for _ in range(4)]
                qq = [jnp.zeros((L16,), jnp.float32) for _ in range(4)]
                for d in range(DIM):
                    cc = (ri + d) & (DIM - 1)
                    v = plsc.load_gather(rows_v, [rows_idx, cc])
                    ss[d % 4] = ss[d % 4] + v
                    qq[d % 4] = qq[d % 4] + v * v
                s = (ss[0] + ss[1]) + (ss[2] + ss[3])
                q = (qq[0] + qq[1]) + (qq[2] + qq[3])
                mean = s * (1.0 / DIM)
                var = q * (1.0 / DIM) - mean * mean
                rstd = _rsqrt(jnp.maximum(var, 0.0) + EPS)
                for d in range(DIM):
                    cc = (ri + d) & (DIM - 1)
                    v = plsc.load_gather(rows_v, [rows_idx, cc])
                    o = (v - mean) * rstd
                    plsc.store_scatter(rows_v, [rows_idx, cc], o)
                for r in range(L16):
                    rr = jnp.full((L16,), 1, jnp.int32) * (blk * L16 + r)
                    for k in range(DIM // L16):
                        cidx = ri + k * L16
                        v = plsc.load_gather(rows_v, [rr, cidx])
                        o = v * wregs[k] + bregs[k]
                        plsc.store_scatter(rows_v, [rr, cidx], o)
                return c2

            lax.fori_loop(0, CHUNK // L16, block_body, 0)
            pltpu.sync_copy(rows_v, out_hbm.at[pl.ds(base + t * CHUNK, CHUNK)])
            return carry

        lax.fori_loop(0, nchunk, chunk_body, 0)

    return sc_kernel


def kernel(input_ids, value_table, ln_weight, ln_bias):
    b, l = input_ids.shape
    ids_flat = input_ids.reshape(-1).astype(jnp.int32)
    out = _build(b * l)(ids_flat, value_table,
                        ln_weight.astype(jnp.float32),
                        ln_bias.astype(jnp.float32))
    return out.reshape(b, l, DIM)
